# baseline (speedup 1.0000x reference)
import functools

import jax
import jax.numpy as jnp
from jax import lax
from jax.experimental import pallas as pl
from jax.experimental.pallas import tpu as pltpu
from jax.experimental.pallas import tpu_sc as plsc


def _i32(x):
    return jnp.asarray(x, jnp.int32)


_NC = 2
_NS = 16
_NW = _NC * _NS
_PC = 128
_NCLS = 10
_NEU = 128
_TS = 16


def _build_w_body(perm_hbm, w_hbm, perm_v, tile_v):
    wid = lax.axis_index("s") * _NC + lax.axis_index("c")
    strip = wid % _i32(16)
    half = wid // _i32(16)
    base = strip * _i32(_PC)

    tvec = lax.iota(jnp.int32, 16)
    wvec = jnp.left_shift(_i32(1), _i32(15) - tvec).astype(jnp.float32)
    zvec = jnp.zeros((16,), jnp.float32)
    lane = lax.iota(jnp.int32, 16) * _i32(0)

    def _zrow(r, _):
        def _zq(q, __):
            plsc.store_scatter(tile_v, [lane + r, lane + q * _i32(16) + tvec],
                               zvec)
            return __
        lax.fori_loop(_i32(0), _i32(_PC // 16), _zq, 0)
        return _
    lax.fori_loop(_i32(0), _i32(_NEU), _zrow, 0)

    for k in range(_NCLS // 2):
        cls = half * _i32(_NCLS // 2) + _i32(k)
        pltpu.sync_copy(perm_hbm.at[cls], perm_v)

        def _scat(j, _):
            v = perm_v[pl.ds(j * _i32(16), 16)]
            mask = (v >= base) & (v < base + _PC)
            plsc.store_scatter(tile_v, [lane + j, v - base], wvec, mask=mask)
            return _
        lax.fori_loop(_i32(0), _i32(_NEU), _scat, 0)

        pltpu.sync_copy(tile_v, w_hbm.at[cls, :, pl.ds(base, _PC)])

        def _unscat(j, _):
            v = perm_v[pl.ds(j * _i32(16), 16)]
            mask = (v >= base) & (v < base + _PC)
            plsc.store_scatter(tile_v, [lane + j, v - base], zvec, mask=mask)
            return _
        lax.fori_loop(_i32(0), _i32(_NEU), _unscat, 0)


def _build_w(perm_i32_2d):
    mesh = plsc.VectorSubcoreMesh(core_axis_name="c", subcore_axis_name="s")
    return pl.kernel(
        _build_w_body,
        mesh=mesh,
        out_type=jax.ShapeDtypeStruct((_NCLS, _NEU, 2048), jnp.float32),
        scratch_types=[
            pltpu.VMEM((2048,), jnp.int32),
            pltpu.VMEM((_NEU, _PC), jnp.float32),
        ],
        compiler_params=pltpu.CompilerParams(needs_layout_passes=False),
    )(perm_i32_2d)


def _rank_body(samples_ref, w_ref, trained_ref, out_ref, addr_ref):
    addr_ref[...] = lax.dot_general(
        w_ref[0], samples_ref[...],
        dimension_numbers=(((1,), (1,)), ((), ())),
        preferred_element_type=jnp.float32,
    ).astype(jnp.int32)

    RC = 8
    cnt = jnp.zeros((RC, 1024), jnp.int32)
    for rc in range(0, 128, RC):
        a = addr_ref[rc:rc + RC, :]
        m = jnp.zeros((RC, 1024), jnp.bool_)
        for k in range(64):
            tk = trained_ref[0, rc:rc + RC, k:k + 1]
            m = m | (a == tk)
        cnt = cnt + m.astype(jnp.int32)

    out_ref[0] = jnp.sum(cnt, axis=0, keepdims=True,
                         dtype=jnp.int32)


def _rank(samples_f32, w, trained_i32):
    n_classes = trained_i32.shape[0]
    return pl.pallas_call(
        _rank_body,
        grid=(n_classes,),
        in_specs=[
            pl.BlockSpec((1024, 2048), lambda i: (_i32(0), _i32(0))),
            pl.BlockSpec((1, 128, 2048), lambda i: (i, _i32(0), _i32(0))),
            pl.BlockSpec((1, 128, 64), lambda i: (i, _i32(0), _i32(0))),
        ],
        out_specs=pl.BlockSpec((1, 1, 1024), lambda i: (i, _i32(0), _i32(0))),
        out_shape=jax.ShapeDtypeStruct((n_classes, 1, 1024), jnp.int32),
        scratch_shapes=[
            pltpu.VMEM((128, 1024), jnp.int32),
        ],
    )(samples_f32, w, trained_i32)


def kernel(samples, tuple_mapping, trained_tuples):
    B, entry_size = samples.shape
    n_classes, n_neurons, K = trained_tuples.shape
    samples_f32 = samples.astype(jnp.int32).astype(jnp.float32)
    perm_i32 = tuple_mapping.astype(jnp.int32)
    trained_i32 = trained_tuples.astype(jnp.int32)
    w = _build_w(perm_i32)
    resp = _rank(samples_f32, w, trained_i32)
    return resp.reshape(n_classes, B).T.astype(jnp.int8)
